# baseline (speedup 1.0000x reference)
import functools

import jax
import jax.numpy as jnp
from jax import lax
from jax.experimental import pallas as pl
from jax.experimental.pallas import tpu as pltpu
from jax.experimental.pallas import tpu_sc as plsc

_N = 1000
_NPAD = 1024
_NCHUNK = _NPAD // 16
_K = 16
_DOUT = 64
_DROW = 65
_FEAT = 18
_BATCH = 32
_NBLK = 5
_RPB = _N // _NBLK


def _merge16(ak, av, bk, bv):
    rk = lax.rev(bk, (0,))
    rv = lax.rev(bv, (0,))
    m = ak <= rk
    nk = jnp.where(m, ak, rk)
    nv = jnp.where(m, av, rv)
    nk, nv = plsc.sort_key_val(nk, nv)
    return nk, nv


def _mlp_tc_kernel(in_ref, wd_ref, w2_ref, b_ref, a_ref, bt_ref):
    x = in_ref[0][:, :_FEAT]
    a_ref[...] = (
        jnp.dot(x, wd_ref[...], preferred_element_type=jnp.float32)
        + b_ref[...]
    )[None]
    bt_ref[...] = jnp.dot(
        x, w2_ref[...], preferred_element_type=jnp.float32)[None]


def _edge_sc_body(x_hbm, y_hbm, a_hbm, b_hbm, out_hbm, xv, yv, bv, av, ov, iv,
                  ska, sva, swk, swi, rank):
    wid = lax.axis_index("s") * 2 + lax.axis_index("c")
    pltpu.sync_copy(x_hbm.at[wid], xv)
    pltpu.sync_copy(y_hbm.at[wid], yv)
    pltpu.sync_copy(b_hbm.at[wid], bv)

    iota = lax.broadcasted_iota(jnp.int32, (16,), 0)
    inf = jnp.float32(jnp.inf)
    init_k = jnp.full((16,), inf, jnp.float32)
    init_v = jnp.zeros((16,), jnp.int32)
    onev = jnp.ones((16,), jnp.float32)

    swk[pl.ds(0, 16)] = jnp.full((16,), -1e30, jnp.float32)
    swi[pl.ds(0, 16)] = jnp.full((16,), _N, jnp.int32)
    swk[pl.ds(_NPAD + 16, 16)] = jnp.full((16,), 1e30, jnp.float32)
    swi[pl.ds(_NPAD + 16, 16)] = jnp.full((16,), _N, jnp.int32)
    for c in range(_NCHUNK):
        k_, v_ = plsc.sort_key_val(xv[pl.ds(c * 16, 16)], iota + c * 16)
        swk[pl.ds(16 + c * 16, 16)] = k_
        swi[pl.ds(16 + c * 16, 16)] = v_

    def _merge_pass(src_k, src_v, soff, dst_k, dst_v, doff, m):
        nblk = (2 * m) // 16
        npairs = _NPAD // (2 * m)

        def pair_body(p, _):
            abase = soff + p * 2 * m
            bbase = abase + m
            pk0 = src_k[pl.ds(abase, 16)]
            pv0 = src_v[pl.ds(abase, 16)]

            def step(t, carry):
                pa, pb, pk, pv = carry
                a_avail = pa < m // 16
                b_avail = pb < m // 16
                ha = plsc.load_gather(
                    src_k, [jnp.full((16,), abase + pa * 16, jnp.int32)])
                hb = plsc.load_gather(
                    src_k, [jnp.full((16,), bbase + pb * 16, jnp.int32)])
                pick_a = jnp.logical_and(
                    a_avail,
                    jnp.logical_or(jnp.logical_not(b_avail),
                                   jnp.any(ha <= hb)))
                nbase = jnp.where(pick_a, abase + pa * 16, bbase + pb * 16)
                ck = src_k[pl.ds(nbase, 16)]
                cv = src_v[pl.ds(nbase, 16)]
                rk = lax.rev(ck, (0,))
                rv = lax.rev(cv, (0,))
                mle = pk <= rk
                bk = jnp.where(mle, pk, rk)
                bvv = jnp.where(mle, pv, rv)
                tk = jnp.where(mle, rk, pk)
                tv = jnp.where(mle, rv, pv)
                bk, bvv = plsc.sort_key_val(bk, bvv)
                tk, tv = plsc.sort_key_val(tk, tv)
                obase = doff - soff + abase + t * 16
                dst_k[pl.ds(obase, 16)] = bk
                dst_v[pl.ds(obase, 16)] = bvv
                pa = pa + pick_a.astype(jnp.int32)
                pb = pb + (1 - pick_a.astype(jnp.int32))
                return pa, pb, tk, tv

            _, _, pk, pv = lax.fori_loop(0, nblk - 1, step, (1, 0, pk0, pv0))
            dst_k[pl.ds(doff - soff + abase + (nblk - 1) * 16, 16)] = pk
            dst_v[pl.ds(doff - soff + abase + (nblk - 1) * 16, 16)] = pv
            return 0

        lax.fori_loop(0, npairs, pair_body, 0)

    _merge_pass(swk, swi, 16, ska, sva, 0, 16)
    _merge_pass(ska, sva, 0, swk, swi, 16, 32)
    _merge_pass(swk, swi, 16, ska, sva, 0, 64)
    _merge_pass(ska, sva, 0, swk, swi, 16, 128)
    _merge_pass(swk, swi, 16, ska, sva, 0, 256)
    _merge_pass(ska, sva, 0, swk, swi, 16, 512)

    for c in range(_NCHUNK):
        ids_c = swi[pl.ds(16 + c * 16, 16)]
        plsc.store_scatter(rank, [ids_c], iota + (16 + c * 16))

    for blk in range(_NBLK):
        pltpu.sync_copy(
            a_hbm.at[wid, pl.ds(blk * _RPB * _DOUT, _RPB * _DOUT)], av)

        def row_body(r, _, blk=blk):
            i = blk * _RPB + r
            ii = jnp.full((16,), i, jnp.int32)
            xi = plsc.load_gather(xv, [ii])
            yi = plsc.load_gather(yv, [ii])

            pxv = plsc.load_gather(rank, [ii])
            xl0 = plsc.load_gather(swk, [pxv - 1])
            xr0 = plsc.load_gather(swk, [pxv])
            dl0 = (xl0 - xi) * (xl0 - xi)
            dr0 = (xr0 - xi) * (xr0 - xi)

            def w_cond(carry):
                lk, lv, wl, wr, dl, dr = carry
                return jnp.logical_or(jnp.any(dl < lk), jnp.any(dr < lk))

            def w_body(carry):
                lk, lv, wl, wr, dl, dr = carry
                go_l = jnp.any(dl < dr)
                base = jnp.where(go_l, wl - 16, wr)
                cidx = base + iota
                ckx = plsc.load_gather(swk, [cidx])
                cid = plsc.load_gather(swi, [cidx])
                cy = plsc.load_gather(yv, [cid])
                dx = ckx - xi
                dy = cy - yi
                d = dx * dx + dy * dy
                d = jnp.where(cid == i, inf, d)
                sk_, sv_ = plsc.sort_key_val(d, cid)
                lk, lv = _merge16(lk, lv, sk_, sv_)
                wl = jnp.where(go_l, wl - 16, wl)
                wr = jnp.where(go_l, wr, wr + 16)
                xlv = plsc.load_gather(swk, [wl - 1])
                xrv = plsc.load_gather(swk, [wr])
                dl = (xlv - xi) * (xlv - xi)
                dr = (xrv - xi) * (xrv - xi)
                return lk, lv, wl, wr, dl, dr

            _, lv, _, _, _, _ = lax.while_loop(
                w_cond, w_body, (init_k, init_v, pxv, pxv, dl0, dr0))
            iv[pl.ds(16, 16)] = lv

            a = [av[pl.ds(r * _DOUT + 16 * c4, 16)] for c4 in range(4)]
            acc = [jnp.zeros((16,), jnp.float32) for _ in range(4)]
            for k in range(_K):
                nb = plsc.load_gather(iv, [jnp.full((16,), 16 + k, jnp.int32)])
                for c4 in range(4):
                    col = iota + 16 * c4
                    bvec = plsc.load_gather(bv, [nb, col])
                    acc[c4] = acc[c4] + jnp.maximum(bvec + a[c4], 0.0)
            scale = jnp.float32(1.0 / _K)
            for c4 in range(4):
                ov[pl.ds(r * _DROW + 16 * c4, 16)] = acc[c4] * scale
            return 0

        lax.fori_loop(0, _RPB, row_body, 0)
        last = _RPB * _DROW - 1
        for t in range((_RPB + 15) // 16):
            idx = jnp.minimum(iota * _DROW + _DOUT + t * 16 * _DROW, last)
            plsc.store_scatter(ov, [idx], onev)
        pltpu.sync_copy(
            ov, out_hbm.at[wid, pl.ds(blk * _RPB * _DROW, _RPB * _DROW)])


@functools.partial(
    pl.kernel,
    out_type=jax.ShapeDtypeStruct((_BATCH, _N * _DROW), jnp.float32),
    mesh=plsc.VectorSubcoreMesh(core_axis_name="c", subcore_axis_name="s"),
    compiler_params=pltpu.CompilerParams(
        needs_layout_passes=False, use_tc_tiling_on_sc=False),
    scratch_types=[
        pltpu.VMEM((_NPAD,), jnp.float32),
        pltpu.VMEM((_NPAD,), jnp.float32),
        pltpu.VMEM((_N, _DOUT), jnp.float32),
        pltpu.VMEM((_RPB * _DOUT,), jnp.float32),
        pltpu.VMEM((_RPB * _DROW,), jnp.float32),
        pltpu.VMEM((32,), jnp.int32),
        pltpu.VMEM((_NPAD,), jnp.float32),
        pltpu.VMEM((_NPAD,), jnp.int32),
        pltpu.VMEM((_NPAD + 32,), jnp.float32),
        pltpu.VMEM((_NPAD + 32,), jnp.int32),
        pltpu.VMEM((_NPAD,), jnp.int32),
    ],
)
def _edge_sc(x_hbm, y_hbm, a_hbm, b_hbm, out_hbm, xv, yv, bv, av, ov, iv,
             ska, sva, swk, swi, rank):
    _edge_sc_body(x_hbm, y_hbm, a_hbm, b_hbm, out_hbm, xv, yv, bv, av, ov, iv,
                  ska, sva, swk, swi, rank)


def kernel(inputs, W, b):
    x = inputs[:, :, 0]
    y = inputs[:, :, 1]
    pad = jnp.full((_BATCH, _NPAD - _N), 1e30, jnp.float32)
    xp = jnp.concatenate([x, pad], axis=1)
    yp = jnp.concatenate([y, pad], axis=1)

    w1 = W[:_FEAT]
    w2 = W[_FEAT:]
    wd = w1 - w2
    b2d = b[None, :]

    a_tab, b_tab = pl.pallas_call(
        _mlp_tc_kernel,
        grid=(_BATCH,),
        in_specs=[
            pl.BlockSpec((1, _N, _FEAT + 1), lambda i: (i, 0, 0)),
            pl.BlockSpec((_FEAT, _DOUT), lambda i: (0, 0)),
            pl.BlockSpec((_FEAT, _DOUT), lambda i: (0, 0)),
            pl.BlockSpec((1, _DOUT), lambda i: (0, 0)),
        ],
        out_specs=[
            pl.BlockSpec((1, _N, _DOUT), lambda i: (i, 0, 0)),
            pl.BlockSpec((1, _N, _DOUT), lambda i: (i, 0, 0)),
        ],
        out_shape=[
            jax.ShapeDtypeStruct((_BATCH, _N, _DOUT), jnp.float32),
            jax.ShapeDtypeStruct((_BATCH, _N, _DOUT), jnp.float32),
        ],
    )(inputs, wd, w2, b2d)

    a_flat = a_tab.reshape(_BATCH, _N * _DOUT)
    out_flat = _edge_sc(xp, yp, a_flat, b_tab)
    return out_flat.reshape(_BATCH, _N, _DROW)


# speedup vs baseline: 2.4459x; 2.4459x over previous
import functools

import jax
import jax.numpy as jnp
from jax import lax
from jax.experimental import pallas as pl
from jax.experimental.pallas import tpu as pltpu
from jax.experimental.pallas import tpu_sc as plsc

_N = 1000
_NPAD = 1024
_NCHUNK = _NPAD // 16
_K = 16
_DOUT = 64
_DROW = 65
_FEAT = 18
_BATCH = 32
_NBLK = 5
_RPB = _N // _NBLK


def _merge16(ak, av, bk, bv):
    rk = lax.rev(bk, (0,))
    rv = lax.rev(bv, (0,))
    m = ak <= rk
    nk = jnp.where(m, ak, rk)
    nv = jnp.where(m, av, rv)
    nk, nv = plsc.sort_key_val(nk, nv)
    return nk, nv


def _mlp_tc_kernel(in_ref, wd_ref, w2_ref, b_ref, a_ref, bt_ref):
    x = in_ref[0][:, :_FEAT]
    a_ref[...] = (
        jnp.dot(x, wd_ref[...], preferred_element_type=jnp.float32)
        + b_ref[...]
    )[None]
    bt_ref[...] = jnp.dot(
        x, w2_ref[...], preferred_element_type=jnp.float32)[None]


def _edge_sc_body(x_hbm, y_hbm, a_hbm, b_hbm, out_hbm, xv, yv, bv, av, ov, iv):
    wid = lax.axis_index("s") * 2 + lax.axis_index("c")
    pltpu.sync_copy(x_hbm.at[wid], xv)
    pltpu.sync_copy(y_hbm.at[wid], yv)
    pltpu.sync_copy(b_hbm.at[wid], bv)

    iota = lax.broadcasted_iota(jnp.int32, (16,), 0)
    inf = jnp.float32(jnp.inf)
    init_k = jnp.full((16,), inf, jnp.float32)
    init_v = jnp.zeros((16,), jnp.int32)
    onev = jnp.ones((16,), jnp.float32)

    for blk in range(_NBLK):
        pltpu.sync_copy(
            a_hbm.at[wid, pl.ds(blk * _RPB * _DOUT, _RPB * _DOUT)], av)

        def row_body(r, _, blk=blk):
            i = blk * _RPB + r
            ii = jnp.full((16,), i, jnp.int32)
            xi = plsc.load_gather(xv, [ii])
            yi = plsc.load_gather(yv, [ii])

            def quad_body(c, carry):
                outs = []
                for t in range(4):
                    lk, lv = carry[2 * t], carry[2 * t + 1]
                    leaves = []
                    for q in range(4):
                        base = t * 256 + c * 64 + q * 16
                        dx = xv[pl.ds(base, 16)] - xi
                        dy = yv[pl.ds(base, 16)] - yi
                        d = dx * dx + dy * dy
                        jc = iota + base
                        d = jnp.where(jc == i, inf, d)
                        leaves.append(plsc.sort_key_val(d, jc))
                    m1 = _merge16(*leaves[0], *leaves[1])
                    m2 = _merge16(*leaves[2], *leaves[3])
                    m3 = _merge16(*m1, *m2)
                    outs.extend(_merge16(lk, lv, *m3))
                return tuple(outs)

            fin = lax.fori_loop(0, 4, quad_body,
                                (init_k, init_v) * 4)
            f1 = _merge16(fin[0], fin[1], fin[2], fin[3])
            f2 = _merge16(fin[4], fin[5], fin[6], fin[7])
            _, lv = _merge16(*f1, *f2)
            iv[pl.ds(16, 16)] = lv

            a = [av[pl.ds(r * _DOUT + 16 * c4, 16)] for c4 in range(4)]
            acc = [jnp.zeros((16,), jnp.float32) for _ in range(4)]
            for k in range(_K):
                nb = plsc.load_gather(iv, [jnp.full((16,), 16 + k, jnp.int32)])
                for c4 in range(4):
                    col = iota + 16 * c4
                    bvec = plsc.load_gather(bv, [nb, col])
                    acc[c4] = acc[c4] + jnp.maximum(bvec + a[c4], 0.0)
            scale = jnp.float32(1.0 / _K)
            for c4 in range(4):
                ov[pl.ds(r * _DROW + 16 * c4, 16)] = acc[c4] * scale
            return 0

        lax.fori_loop(0, _RPB, row_body, 0)
        last = _RPB * _DROW - 1
        for t in range((_RPB + 15) // 16):
            idx = jnp.minimum(iota * _DROW + _DOUT + t * 16 * _DROW, last)
            plsc.store_scatter(ov, [idx], onev)
        pltpu.sync_copy(
            ov, out_hbm.at[wid, pl.ds(blk * _RPB * _DROW, _RPB * _DROW)])


@functools.partial(
    pl.kernel,
    out_type=jax.ShapeDtypeStruct((_BATCH, _N * _DROW), jnp.float32),
    mesh=plsc.VectorSubcoreMesh(core_axis_name="c", subcore_axis_name="s"),
    compiler_params=pltpu.CompilerParams(
        needs_layout_passes=False, use_tc_tiling_on_sc=False),
    scratch_types=[
        pltpu.VMEM((_NPAD,), jnp.float32),
        pltpu.VMEM((_NPAD,), jnp.float32),
        pltpu.VMEM((_N, _DOUT), jnp.float32),
        pltpu.VMEM((_RPB * _DOUT,), jnp.float32),
        pltpu.VMEM((_RPB * _DROW,), jnp.float32),
        pltpu.VMEM((32,), jnp.int32),
    ],
)
def _edge_sc(x_hbm, y_hbm, a_hbm, b_hbm, out_hbm, xv, yv, bv, av, ov, iv):
    _edge_sc_body(x_hbm, y_hbm, a_hbm, b_hbm, out_hbm, xv, yv, bv, av, ov, iv)


def kernel(inputs, W, b):
    x = inputs[:, :, 0]
    y = inputs[:, :, 1]
    pad = jnp.full((_BATCH, _NPAD - _N), 1e30, jnp.float32)
    xp = jnp.concatenate([x, pad], axis=1)
    yp = jnp.concatenate([y, pad], axis=1)

    w1 = W[:_FEAT]
    w2 = W[_FEAT:]
    wd = w1 - w2
    b2d = b[None, :]

    a_tab, b_tab = pl.pallas_call(
        _mlp_tc_kernel,
        grid=(_BATCH,),
        in_specs=[
            pl.BlockSpec((1, _N, _FEAT + 1), lambda i: (i, 0, 0)),
            pl.BlockSpec((_FEAT, _DOUT), lambda i: (0, 0)),
            pl.BlockSpec((_FEAT, _DOUT), lambda i: (0, 0)),
            pl.BlockSpec((1, _DOUT), lambda i: (0, 0)),
        ],
        out_specs=[
            pl.BlockSpec((1, _N, _DOUT), lambda i: (i, 0, 0)),
            pl.BlockSpec((1, _N, _DOUT), lambda i: (i, 0, 0)),
        ],
        out_shape=[
            jax.ShapeDtypeStruct((_BATCH, _N, _DOUT), jnp.float32),
            jax.ShapeDtypeStruct((_BATCH, _N, _DOUT), jnp.float32),
        ],
    )(inputs, wd, w2, b2d)

    a_flat = a_tab.reshape(_BATCH, _N * _DOUT)
    out_flat = _edge_sc(xp, yp, a_flat, b_tab)
    return out_flat.reshape(_BATCH, _N, _DROW)


# speedup vs baseline: 2.5336x; 1.0359x over previous
import functools

import jax
import jax.numpy as jnp
from jax import lax
from jax.experimental import pallas as pl
from jax.experimental.pallas import tpu as pltpu
from jax.experimental.pallas import tpu_sc as plsc

_N = 1000
_NPAD = 1024
_NCHUNK = _NPAD // 16
_K = 16
_DOUT = 64
_DROW = 65
_FEAT = 18
_BATCH = 32
_NBLK = 5
_RPB = _N // _NBLK


def _merge16(ak, av, bk, bv):
    rk = lax.rev(bk, (0,))
    rv = lax.rev(bv, (0,))
    m = ak <= rk
    nk = jnp.where(m, ak, rk)
    nv = jnp.where(m, av, rv)
    nk, nv = plsc.sort_key_val(nk, nv)
    return nk, nv


def _mlp_tc_kernel(in_ref, wd_ref, w2_ref, b_ref, a_ref, bt_ref):
    x = in_ref[0][:, :_FEAT]
    a_ref[...] = (
        jnp.dot(x, wd_ref[...], preferred_element_type=jnp.float32)
        + b_ref[...]
    )[None]
    bt_ref[...] = jnp.dot(
        x, w2_ref[...], preferred_element_type=jnp.float32)[None]


def _edge_sc_body(x_hbm, y_hbm, a_hbm, b_hbm, out_hbm, xv, yv, bv, av, ov, iv):
    wid = lax.axis_index("s") * 2 + lax.axis_index("c")
    pltpu.sync_copy(x_hbm.at[wid], xv)
    pltpu.sync_copy(y_hbm.at[wid], yv)
    pltpu.sync_copy(b_hbm.at[wid], bv)

    iota = lax.broadcasted_iota(jnp.int32, (16,), 0)
    inf = jnp.float32(jnp.inf)
    init_k = jnp.full((16,), inf, jnp.float32)
    init_v = jnp.zeros((16,), jnp.int32)
    onev = jnp.ones((16,), jnp.float32)

    for blk in range(_NBLK):
        pltpu.sync_copy(
            a_hbm.at[wid, pl.ds(blk * _RPB * _DOUT, _RPB * _DOUT)], av)

        def row_body(r, _, blk=blk):
            i = blk * _RPB + r
            ii = jnp.full((16,), i, jnp.int32)
            xi = plsc.load_gather(xv, [ii])
            yi = plsc.load_gather(yv, [ii])

            def quad_body(c, carry):
                lk, lv = carry
                leaves = []
                for q in range(4):
                    base = c * 64 + q * 16
                    dx = xv[pl.ds(base, 16)] - xi
                    dy = yv[pl.ds(base, 16)] - yi
                    d = dx * dx + dy * dy
                    jc = iota + base
                    d = jnp.where(jc == i, inf, d)
                    leaves.append(plsc.sort_key_val(d, jc))
                m1 = _merge16(*leaves[0], *leaves[1])
                m2 = _merge16(*leaves[2], *leaves[3])
                m3 = _merge16(*m1, *m2)
                return _merge16(lk, lv, *m3)

            _, lv = lax.fori_loop(0, _NCHUNK // 4, quad_body,
                                  (init_k, init_v))

            a = [av[pl.ds(r * _DOUT + 16 * c4, 16)] for c4 in range(4)]
            acc = [jnp.zeros((16,), jnp.float32) for _ in range(4)]
            for k in range(_K):
                nb = lax.gather(
                    lv, jnp.full((16, 1), k, jnp.int32),
                    dimension_numbers=lax.GatherDimensionNumbers(
                        offset_dims=(), collapsed_slice_dims=(0,),
                        start_index_map=(0,)),
                    slice_sizes=(1,),
                    mode=lax.GatherScatterMode.PROMISE_IN_BOUNDS)
                for c4 in range(4):
                    col = iota + 16 * c4
                    bvec = plsc.load_gather(bv, [nb, col])
                    acc[c4] = acc[c4] + jnp.maximum(bvec + a[c4], 0.0)
            scale = jnp.float32(1.0 / _K)
            for c4 in range(4):
                ov[pl.ds(r * _DROW + 16 * c4, 16)] = acc[c4] * scale
            return 0

        lax.fori_loop(0, _RPB, row_body, 0)
        last = _RPB * _DROW - 1
        for t in range((_RPB + 15) // 16):
            idx = jnp.minimum(iota * _DROW + _DOUT + t * 16 * _DROW, last)
            plsc.store_scatter(ov, [idx], onev)
        pltpu.sync_copy(
            ov, out_hbm.at[wid, pl.ds(blk * _RPB * _DROW, _RPB * _DROW)])


@functools.partial(
    pl.kernel,
    out_type=jax.ShapeDtypeStruct((_BATCH, _N * _DROW), jnp.float32),
    mesh=plsc.VectorSubcoreMesh(core_axis_name="c", subcore_axis_name="s"),
    compiler_params=pltpu.CompilerParams(
        needs_layout_passes=False, use_tc_tiling_on_sc=False),
    scratch_types=[
        pltpu.VMEM((_NPAD,), jnp.float32),
        pltpu.VMEM((_NPAD,), jnp.float32),
        pltpu.VMEM((_N, _DOUT), jnp.float32),
        pltpu.VMEM((_RPB * _DOUT,), jnp.float32),
        pltpu.VMEM((_RPB * _DROW,), jnp.float32),
        pltpu.VMEM((32,), jnp.int32),
    ],
)
def _edge_sc(x_hbm, y_hbm, a_hbm, b_hbm, out_hbm, xv, yv, bv, av, ov, iv):
    _edge_sc_body(x_hbm, y_hbm, a_hbm, b_hbm, out_hbm, xv, yv, bv, av, ov, iv)


def kernel(inputs, W, b):
    x = inputs[:, :, 0]
    y = inputs[:, :, 1]
    pad = jnp.full((_BATCH, _NPAD - _N), 1e30, jnp.float32)
    xp = jnp.concatenate([x, pad], axis=1)
    yp = jnp.concatenate([y, pad], axis=1)

    w1 = W[:_FEAT]
    w2 = W[_FEAT:]
    wd = w1 - w2
    b2d = b[None, :]

    a_tab, b_tab = pl.pallas_call(
        _mlp_tc_kernel,
        grid=(_BATCH,),
        in_specs=[
            pl.BlockSpec((1, _N, _FEAT + 1), lambda i: (i, 0, 0)),
            pl.BlockSpec((_FEAT, _DOUT), lambda i: (0, 0)),
            pl.BlockSpec((_FEAT, _DOUT), lambda i: (0, 0)),
            pl.BlockSpec((1, _DOUT), lambda i: (0, 0)),
        ],
        out_specs=[
            pl.BlockSpec((1, _N, _DOUT), lambda i: (i, 0, 0)),
            pl.BlockSpec((1, _N, _DOUT), lambda i: (i, 0, 0)),
        ],
        out_shape=[
            jax.ShapeDtypeStruct((_BATCH, _N, _DOUT), jnp.float32),
            jax.ShapeDtypeStruct((_BATCH, _N, _DOUT), jnp.float32),
        ],
    )(inputs, wd, w2, b2d)

    a_flat = a_tab.reshape(_BATCH, _N * _DOUT)
    out_flat = _edge_sc(xp, yp, a_flat, b_tab)
    return out_flat.reshape(_BATCH, _N, _DROW)


# speedup vs baseline: 2.5352x; 1.0006x over previous
import functools

import jax
import jax.numpy as jnp
from jax import lax
from jax.experimental import pallas as pl
from jax.experimental.pallas import tpu as pltpu
from jax.experimental.pallas import tpu_sc as plsc

_N = 1000
_NPAD = 1024
_NCHUNK = _NPAD // 16
_K = 16
_DOUT = 64
_DROW = 65
_FEAT = 18
_BATCH = 32
_NBLK = 5
_RPB = _N // _NBLK


def _merge16(ak, av, bk, bv):
    rk = lax.rev(bk, (0,))
    rv = lax.rev(bv, (0,))
    m = ak <= rk
    nk = jnp.where(m, ak, rk)
    nv = jnp.where(m, av, rv)
    nk, nv = plsc.sort_key_val(nk, nv)
    return nk, nv


def _mlp_tc_kernel(in_ref, wd_ref, w2_ref, b_ref, a_ref, bt_ref):
    x = in_ref[0][:, :_FEAT]
    a_ref[...] = (
        jnp.dot(x, wd_ref[...], preferred_element_type=jnp.float32)
        + b_ref[...]
    )[None]
    bt_ref[...] = jnp.dot(
        x, w2_ref[...], preferred_element_type=jnp.float32)[None]


def _edge_sc_body(x_hbm, y_hbm, a_hbm, b_hbm, out_hbm, xv, yv, bv, av, ov):
    wid = lax.axis_index("s") * 2 + lax.axis_index("c")
    pltpu.sync_copy(x_hbm.at[wid], xv)
    pltpu.sync_copy(y_hbm.at[wid], yv)
    pltpu.sync_copy(b_hbm.at[wid], bv)

    iota = lax.broadcasted_iota(jnp.int32, (16,), 0)
    inf = jnp.float32(jnp.inf)
    init_k = jnp.full((16,), inf, jnp.float32)
    init_v = jnp.zeros((16,), jnp.int32)
    onev = jnp.ones((16,), jnp.float32)

    for blk in range(_NBLK):
        pltpu.sync_copy(
            a_hbm.at[wid, pl.ds(blk * _RPB * _DOUT, _RPB * _DOUT)], av)

        def row_body(r, _, blk=blk):
            i = blk * _RPB + r
            ii = jnp.full((16,), i, jnp.int32)
            xi = plsc.load_gather(xv, [ii])
            yi = plsc.load_gather(yv, [ii])

            def quad_body(c, carry):
                lk, lv = carry
                leaves = []
                for q in range(4):
                    base = c * 64 + q * 16
                    dx = xv[pl.ds(base, 16)] - xi
                    dy = yv[pl.ds(base, 16)] - yi
                    d = dx * dx + dy * dy
                    jc = iota + base
                    d = jnp.where(jc == i, inf, d)
                    leaves.append(plsc.sort_key_val(d, jc))
                m1 = _merge16(*leaves[0], *leaves[1])
                m2 = _merge16(*leaves[2], *leaves[3])
                m3 = _merge16(*m1, *m2)
                return _merge16(lk, lv, *m3)

            _, lv = lax.fori_loop(0, _NCHUNK // 4, quad_body,
                                  (init_k, init_v))

            a = [av[pl.ds(r * _DOUT + 16 * c4, 16)] for c4 in range(4)]
            acc = [jnp.zeros((16,), jnp.float32) for _ in range(4)]
            for k in range(_K):
                nb = lax.gather(
                    lv, jnp.full((16, 1), k, jnp.int32),
                    dimension_numbers=lax.GatherDimensionNumbers(
                        offset_dims=(), collapsed_slice_dims=(0,),
                        start_index_map=(0,)),
                    slice_sizes=(1,),
                    mode=lax.GatherScatterMode.PROMISE_IN_BOUNDS)
                for c4 in range(4):
                    col = iota + 16 * c4
                    bvec = plsc.load_gather(bv, [nb, col])
                    acc[c4] = acc[c4] + jnp.maximum(bvec + a[c4], 0.0)
            scale = jnp.float32(1.0 / _K)
            for c4 in range(4):
                ov[pl.ds(r * _DROW + 16 * c4, 16)] = acc[c4] * scale
            return 0

        lax.fori_loop(0, _RPB, row_body, 0)
        last = _RPB * _DROW - 1
        for t in range((_RPB + 15) // 16):
            idx = jnp.minimum(iota * _DROW + _DOUT + t * 16 * _DROW, last)
            plsc.store_scatter(ov, [idx], onev)
        pltpu.sync_copy(
            ov, out_hbm.at[wid, pl.ds(blk * _RPB * _DROW, _RPB * _DROW)])


@functools.partial(
    pl.kernel,
    out_type=jax.ShapeDtypeStruct((_BATCH, _N * _DROW), jnp.float32),
    mesh=plsc.VectorSubcoreMesh(core_axis_name="c", subcore_axis_name="s"),
    compiler_params=pltpu.CompilerParams(
        needs_layout_passes=False, use_tc_tiling_on_sc=False),
    scratch_types=[
        pltpu.VMEM((_NPAD,), jnp.float32),
        pltpu.VMEM((_NPAD,), jnp.float32),
        pltpu.VMEM((_N, _DOUT), jnp.float32),
        pltpu.VMEM((_RPB * _DOUT,), jnp.float32),
        pltpu.VMEM((_RPB * _DROW,), jnp.float32),
    ],
)
def _edge_sc(x_hbm, y_hbm, a_hbm, b_hbm, out_hbm, xv, yv, bv, av, ov):
    _edge_sc_body(x_hbm, y_hbm, a_hbm, b_hbm, out_hbm, xv, yv, bv, av, ov)


def kernel(inputs, W, b):
    x = inputs[:, :, 0]
    y = inputs[:, :, 1]
    pad = jnp.full((_BATCH, _NPAD - _N), 1e30, jnp.float32)
    xp = jnp.concatenate([x, pad], axis=1)
    yp = jnp.concatenate([y, pad], axis=1)

    w1 = W[:_FEAT]
    w2 = W[_FEAT:]
    wd = w1 - w2
    b2d = b[None, :]

    a_tab, b_tab = pl.pallas_call(
        _mlp_tc_kernel,
        grid=(_BATCH,),
        in_specs=[
            pl.BlockSpec((1, _N, _FEAT + 1), lambda i: (i, 0, 0)),
            pl.BlockSpec((_FEAT, _DOUT), lambda i: (0, 0)),
            pl.BlockSpec((_FEAT, _DOUT), lambda i: (0, 0)),
            pl.BlockSpec((1, _DOUT), lambda i: (0, 0)),
        ],
        out_specs=[
            pl.BlockSpec((1, _N, _DOUT), lambda i: (i, 0, 0)),
            pl.BlockSpec((1, _N, _DOUT), lambda i: (i, 0, 0)),
        ],
        out_shape=[
            jax.ShapeDtypeStruct((_BATCH, _N, _DOUT), jnp.float32),
            jax.ShapeDtypeStruct((_BATCH, _N, _DOUT), jnp.float32),
        ],
    )(inputs, wd, w2, b2d)

    a_flat = a_tab.reshape(_BATCH, _N * _DOUT)
    out_flat = _edge_sc(xp, yp, a_flat, b_tab)
    return out_flat.reshape(_BATCH, _N, _DROW)
